# Initial kernel scaffold; baseline (speedup 1.0000x reference)
import jax
import jax.numpy as jnp
from jax.experimental import pallas as pl


def kernel(inputs, V):
    raise NotImplementedError("write your pallas kernel here")


# speedup vs baseline: 4.0005x; 4.0005x over previous
import functools

import jax
import jax.numpy as jnp
from jax import lax
from jax.experimental import pallas as pl
from jax.experimental.pallas import tpu as pltpu
from jax.experimental.pallas import tpu_sc as plsc

NUM_FIELD = 39
NUM_CONT = 13
NUM_CAT = NUM_FIELD - NUM_CONT
EMB = 16
BATCH = 16384
LANES = 16

NUM_CORES = 2
NUM_SUBCORES = 16
NW = NUM_CORES * NUM_SUBCORES
ROWS_PER_W = BATCH // NW
CHUNK = 64
NCHUNKS = ROWS_PER_W // CHUNK
GROUPS = CHUNK // LANES

_mesh = plsc.VectorSubcoreMesh(core_axis_name="c", subcore_axis_name="s")


@functools.partial(
    pl.kernel,
    mesh=_mesh,
    compiler_params=pltpu.CompilerParams(needs_layout_passes=False),
    out_type=jax.ShapeDtypeStruct((BATCH * NUM_FIELD * EMB,), jnp.float32),
    scratch_types=[
        pltpu.VMEM((NUM_FIELD * EMB,), jnp.float32),
        pltpu.VMEM((CHUNK * NUM_FIELD,), jnp.float32),
        pltpu.VMEM((CHUNK * NUM_FIELD * EMB,), jnp.float32),
        pltpu.VMEM((NUM_CAT * LANES,), jnp.int32),
        pltpu.VMEM((NUM_CAT * LANES,), jnp.int32),
    ],
)
def _emb_sc(in_hbm, v_hbm, out_hbm, v_v, in_v, out_v, as_v, ms_v):
    wid = lax.axis_index("s") * NUM_CORES + lax.axis_index("c")
    iota = lax.iota(jnp.int32, LANES)
    pltpu.sync_copy(v_hbm.at[pl.ds(0, NUM_FIELD * EMB)], v_v)

    def scatter_16cols(x, obase, vbase):
        for c in range(EMB):
            vs = plsc.load_gather(v_v, [jnp.broadcast_to(vbase + c, (LANES,)).astype(jnp.int32)])
            plsc.store_scatter(out_v, [obase + c], x * vs)

    def chunk_body(ci, _):
        base = wid * ROWS_PER_W + ci * CHUNK
        pltpu.sync_copy(
            in_hbm.at[pl.ds(base * NUM_FIELD, CHUNK * NUM_FIELD)], in_v)

        def group_body(g, _):
            rows = g * LANES + iota
            in_base = rows * NUM_FIELD
            ob_rows = in_base * EMB

            def cont_body(j, _):
                x = plsc.load_gather(in_v, [in_base + j])
                scatter_16cols(x, ob_rows + j * EMB, j * EMB)
                return 0

            lax.fori_loop(0, NUM_CONT, cont_body, 0)

            def cat1_body(j, carry):
                c_true, c_false = carry
                x = plsc.load_gather(in_v, [in_base + j])
                mi = (x != 0.0).astype(jnp.int32)
                a = jnp.where(mi == 1, c_true + NUM_CONT, c_false)
                sidx = (j - NUM_CONT) * LANES + iota
                plsc.store_scatter(as_v, [sidx], a)
                plsc.store_scatter(ms_v, [sidx], mi)
                return (c_true + mi, c_false + (1 - mi))

            zero = jnp.zeros((LANES,), jnp.int32)
            _, n_false = lax.fori_loop(
                NUM_CONT, NUM_FIELD, cat1_body, (zero, zero))

            def cat2_body(jj, _):
                sidx = jj * LANES + iota
                a = plsc.load_gather(as_v, [sidx])
                mi = plsc.load_gather(ms_v, [sidx])
                x = plsc.load_gather(in_v, [in_base + (jj + NUM_CONT)])
                rank = a + (1 - mi) * (NUM_FIELD - n_false)
                scatter_16cols(x, ob_rows + rank * EMB, (jj + NUM_CONT) * EMB)
                return 0

            lax.fori_loop(0, NUM_CAT, cat2_body, 0)
            return 0

        lax.fori_loop(0, GROUPS, group_body, 0)
        pltpu.sync_copy(
            out_v,
            out_hbm.at[pl.ds(base * NUM_FIELD * EMB, CHUNK * NUM_FIELD * EMB)])
        return 0

    lax.fori_loop(0, NCHUNKS, chunk_body, 0)


def kernel(inputs, V):
    out_flat = _emb_sc(inputs.reshape(-1), V.reshape(-1))
    return out_flat.reshape(BATCH, NUM_FIELD, EMB)
